# Initial kernel scaffold; baseline (speedup 1.0000x reference)
import jax
import jax.numpy as jnp
from jax.experimental import pallas as pl


def kernel(x, pos_table):
    raise NotImplementedError("write your pallas kernel here")


# speedup vs baseline: 1.2556x; 1.2556x over previous
import functools

import jax
import jax.numpy as jnp
from jax import lax
from jax.experimental import pallas as pl
from jax.experimental.pallas import tpu as pltpu
from jax.experimental.pallas import tpu_sc as plsc

_NC = 2
_NS = 16
_NW = _NC * _NS
_L = 16
_CH = 8
_NBUF = 4


def kernel(x, pos_table):
    B, T = x.shape
    H = pos_table.shape[1]
    rows_w = T // _NW
    nch = rows_w // _CH
    mesh = plsc.VectorSubcoreMesh(core_axis_name="c", subcore_axis_name="s")

    @functools.partial(
        pl.kernel,
        out_type=jax.ShapeDtypeStruct((T, H), jnp.float32),
        mesh=mesh,
        scratch_types=[
            pltpu.VMEM((H,), jnp.float32),
            [pltpu.VMEM((_CH, H), jnp.float32)] * _NBUF,
            [pltpu.SemaphoreType.DMA] * _NBUF,
            [pltpu.SemaphoreType.DMA] * _NBUF,
        ],
    )
    def sc_add(x_hbm, pos_hbm, out_hbm, xv, bufs, gsems, ssems):
        wid = lax.axis_index("s") * _NC + lax.axis_index("c")
        base = wid * rows_w
        pltpu.sync_copy(x_hbm, xv)

        def gather(c):
            b = c % _NBUF
            return pltpu.make_async_copy(
                pos_hbm.at[pl.ds(base + c * _CH, _CH)], bufs[b], gsems[b])

        def scatter(c):
            b = c % _NBUF
            return pltpu.make_async_copy(
                bufs[b], out_hbm.at[pl.ds(base + c * _CH, _CH)], ssems[b])

        gather(0).start()
        gather(1).start()
        for c in range(nch):
            b = c % _NBUF
            gather(c).wait()

            def panel_body(jp, carry):
                col0 = jp * (8 * _L)
                xvals = [xv[pl.ds(col0 + k * _L, _L)] for k in range(8)]
                for r in range(_CH):
                    for k in range(8):
                        plsc.addupdate(
                            bufs[b].at[r, pl.ds(col0 + k * _L, _L)],
                            xvals[k])
                return carry

            lax.fori_loop(0, H // (8 * _L), panel_body, 0)
            scatter(c).start()
            if c + 2 < nch:
                if c - 2 >= 0:
                    scatter(c - 2).wait()
                gather(c + 2).start()
        for c in range(nch - 4, nch):
            scatter(c).wait()

    out = sc_add(x.reshape(T), pos_table)
    return out[None]


# speedup vs baseline: 3.4088x; 2.7148x over previous
import jax
import jax.numpy as jnp
from jax.experimental import pallas as pl


def _body(x_ref, pos_ref, o_ref):
    o_ref[...] = pos_ref[...] + x_ref[...]


def kernel(x, pos_table):
    B, T = x.shape
    H = pos_table.shape[1]
    BT = 256
    out = pl.pallas_call(
        _body,
        grid=(T // BT,),
        in_specs=[
            pl.BlockSpec((1, H), lambda i: (0, 0)),
            pl.BlockSpec((BT, H), lambda i: (i, 0)),
        ],
        out_specs=pl.BlockSpec((BT, H), lambda i: (i, 0)),
        out_shape=jax.ShapeDtypeStruct((T, H), jnp.float32),
    )(x, pos_table)
    return out[None]
